# Initial kernel scaffold; baseline (speedup 1.0000x reference)
import jax
import jax.numpy as jnp
from jax.experimental import pallas as pl


def kernel(x, emb, W1, b1, W2, b2):
    raise NotImplementedError("write your pallas kernel here")


# speedup vs baseline: 6.4515x; 6.4515x over previous
import functools

import jax
import jax.numpy as jnp
from jax import lax
from jax.experimental import pallas as pl
from jax.experimental.pallas import tpu as pltpu
from jax.experimental.pallas import tpu_sc as plsc

VOCAB = 30522
EMBED = 128
HIDDEN = 256
NUM_CLASSES = 16
B, L = 4096, 200

NC, NS, LANES = 2, 16, 16
NW = NC * NS
B_PER_W = B // NW
VPR = EMBED // LANES
CHUNKS = ((0, 120), (120, 80))


def _pool_body(x_hbm, emb_hbm, out_hbm, idx_v, rows_v, outb_v, sem):
    wid = lax.axis_index("s") * NC + lax.axis_index("c")
    base = wid * B_PER_W

    def row_body(i, _):
        b = base + i
        pltpu.sync_copy(x_hbm.at[pl.ds(b * L, L)], idx_v)
        cps = [
            pltpu.async_copy(
                emb_hbm.at[idx_v.at[pl.ds(off, n)]],
                rows_v.at[pl.ds(off, n)],
                sem,
            )
            for off, n in CHUNKS
        ]
        for cp in cps:
            cp.wait()

        def red(r, accs):
            return tuple(
                accs[c] + rows_v[r, pl.ds(c * LANES, LANES)] for c in range(VPR)
            )

        accs = lax.fori_loop(
            0, L, red, tuple(jnp.zeros((LANES,), jnp.float32) for _ in range(VPR))
        )
        for c in range(VPR):
            outb_v[i, pl.ds(c * LANES, LANES)] = accs[c] * (1.0 / L)
        return 0

    lax.fori_loop(0, B_PER_W, row_body, 0)
    pltpu.sync_copy(outb_v, out_hbm.at[pl.ds(base, B_PER_W)])


@functools.partial(jax.jit, static_argnames=())
def _sc_pool(x_flat, emb):
    mesh = plsc.VectorSubcoreMesh(
        core_axis_name="c", subcore_axis_name="s", num_cores=NC, num_subcores=NS
    )
    return pl.kernel(
        _pool_body,
        out_type=jax.ShapeDtypeStruct((B, EMBED), jnp.float32),
        mesh=mesh,
        scratch_types=[
            pltpu.VMEM((L,), jnp.int32),
            pltpu.VMEM((L, EMBED), jnp.float32),
            pltpu.VMEM((B_PER_W, EMBED), jnp.float32),
            pltpu.SemaphoreType.DMA,
        ],
    )(x_flat, emb)


def _mlp_body(p_ref, w1_ref, b1_ref, w2_ref, b2_ref, o_ref):
    h = jnp.dot(p_ref[...], w1_ref[...], preferred_element_type=jnp.float32)
    h = jnp.maximum(h + b1_ref[...], 0.0)
    o_ref[...] = (
        jnp.dot(h, w2_ref[...], preferred_element_type=jnp.float32) + b2_ref[...]
    )


def _tc_mlp(pooled, W1, b1, W2, b2):
    blk = 512
    grid = B // blk
    return pl.pallas_call(
        _mlp_body,
        grid=(grid,),
        in_specs=[
            pl.BlockSpec((blk, EMBED), lambda i: (i, 0)),
            pl.BlockSpec((EMBED, HIDDEN), lambda i: (0, 0)),
            pl.BlockSpec((1, HIDDEN), lambda i: (0, 0)),
            pl.BlockSpec((HIDDEN, NUM_CLASSES), lambda i: (0, 0)),
            pl.BlockSpec((1, NUM_CLASSES), lambda i: (0, 0)),
        ],
        out_specs=pl.BlockSpec((blk, NUM_CLASSES), lambda i: (i, 0)),
        out_shape=jax.ShapeDtypeStruct((B, NUM_CLASSES), jnp.float32),
    )(pooled, W1, b1.reshape(1, HIDDEN), W2, b2.reshape(1, NUM_CLASSES))


def kernel(x, emb, W1, b1, W2, b2):
    x_flat = x.reshape(-1).astype(jnp.int32)
    pooled = _sc_pool(x_flat, emb)
    return _tc_mlp(pooled, W1, b1, W2, b2)


# speedup vs baseline: 12.7839x; 1.9815x over previous
import jax
import jax.numpy as jnp
from jax import lax
from jax.experimental import pallas as pl
from jax.experimental.pallas import tpu as pltpu
from jax.experimental.pallas import tpu_sc as plsc

VOCAB = 30522
EMBED = 128
HIDDEN = 256
NUM_CLASSES = 16
B, L = 4096, 200

NC, NS, LANES = 2, 16, 16
NW = NC * NS
B_PER_W = B // NW
VPR = EMBED // LANES
CHUNKS = ((0, 120), (120, 80))
RED_UNROLL = 4


def _pool_body(x_hbm, emb_hbm, out_hbm, idx_v, rows0, rows1, outb_v, sem0, sem1):
    wid = lax.axis_index("s") * NC + lax.axis_index("c")
    base = wid * B_PER_W

    pltpu.sync_copy(x_hbm.at[pl.ds(base * L, B_PER_W * L)], idx_v)

    def fire(rowbuf, sem, r):
        for off, n in CHUNKS:
            pltpu.async_copy(
                emb_hbm.at[idx_v.at[pl.ds(r * L + off, n)]],
                rowbuf.at[pl.ds(off, n)],
                sem,
            )

    def drain(rowbuf, sem, r):
        for off, n in CHUNKS:
            pltpu.make_async_copy(
                emb_hbm.at[idx_v.at[pl.ds(r * L + off, n)]],
                rowbuf.at[pl.ds(off, n)],
                sem,
            ).wait()

    def reduce(rowbuf, r):
        def red(k, accs):
            new = accs
            for u in range(RED_UNROLL):
                row = k * RED_UNROLL + u
                new = tuple(
                    new[c] + rowbuf[row, pl.ds(c * LANES, LANES)]
                    for c in range(VPR)
                )
            return new

        accs = lax.fori_loop(
            0,
            L // RED_UNROLL,
            red,
            tuple(jnp.zeros((LANES,), jnp.float32) for _ in range(VPR)),
        )
        for c in range(VPR):
            outb_v[r, pl.ds(c * LANES, LANES)] = accs[c] * (1.0 / L)

    fire(rows0, sem0, 0)

    def body(i, _):
        r0 = 2 * i
        fire(rows1, sem1, r0 + 1)
        drain(rows0, sem0, r0)
        reduce(rows0, r0)

        @pl.when(r0 + 2 < B_PER_W)
        def _():
            fire(rows0, sem0, r0 + 2)

        drain(rows1, sem1, r0 + 1)
        reduce(rows1, r0 + 1)
        return 0

    lax.fori_loop(0, B_PER_W // 2, body, 0)
    pltpu.sync_copy(outb_v, out_hbm.at[pl.ds(base, B_PER_W)])


def _sc_pool(x_flat, emb):
    mesh = plsc.VectorSubcoreMesh(
        core_axis_name="c", subcore_axis_name="s", num_cores=NC, num_subcores=NS
    )
    return pl.kernel(
        _pool_body,
        out_type=jax.ShapeDtypeStruct((B, EMBED), jnp.float32),
        mesh=mesh,
        scratch_types=[
            pltpu.VMEM((B_PER_W * L,), jnp.int32),
            pltpu.VMEM((L, EMBED), jnp.float32),
            pltpu.VMEM((L, EMBED), jnp.float32),
            pltpu.VMEM((B_PER_W, EMBED), jnp.float32),
            pltpu.SemaphoreType.DMA,
            pltpu.SemaphoreType.DMA,
        ],
    )(x_flat, emb)


def _mlp_body(p_ref, w1_ref, b1_ref, w2_ref, b2_ref, o_ref):
    h = jnp.dot(p_ref[...], w1_ref[...], preferred_element_type=jnp.float32)
    h = jnp.maximum(h + b1_ref[...], 0.0)
    o_ref[...] = (
        jnp.dot(h, w2_ref[...], preferred_element_type=jnp.float32) + b2_ref[...]
    )


def _tc_mlp(pooled, W1, b1, W2, b2):
    blk = 512
    grid = B // blk
    return pl.pallas_call(
        _mlp_body,
        grid=(grid,),
        in_specs=[
            pl.BlockSpec((blk, EMBED), lambda i: (i, 0)),
            pl.BlockSpec((EMBED, HIDDEN), lambda i: (0, 0)),
            pl.BlockSpec((1, HIDDEN), lambda i: (0, 0)),
            pl.BlockSpec((HIDDEN, NUM_CLASSES), lambda i: (0, 0)),
            pl.BlockSpec((1, NUM_CLASSES), lambda i: (0, 0)),
        ],
        out_specs=pl.BlockSpec((blk, NUM_CLASSES), lambda i: (i, 0)),
        out_shape=jax.ShapeDtypeStruct((B, NUM_CLASSES), jnp.float32),
    )(pooled, W1, b1.reshape(1, HIDDEN), W2, b2.reshape(1, NUM_CLASSES))


def kernel(x, emb, W1, b1, W2, b2):
    x_flat = x.reshape(-1).astype(jnp.int32)
    pooled = _sc_pool(x_flat, emb)
    return _tc_mlp(pooled, W1, b1, W2, b2)


# speedup vs baseline: 15.4934x; 1.2119x over previous
import jax
import jax.numpy as jnp
from jax import lax
from jax.experimental import pallas as pl
from jax.experimental.pallas import tpu as pltpu
from jax.experimental.pallas import tpu_sc as plsc

VOCAB = 30522
EMBED = 128
HIDDEN = 256
NUM_CLASSES = 16
B, L = 4096, 200

NC, NS, LANES = 2, 16, 16
NW = NC * NS
B_PER_W = B // NW
VPR = EMBED // LANES
CHUNKS = ((0, 120), (120, 80))
RED_UNROLL = 4


def _pool_body(
    x_hbm, emb_hbm, out_hbm, idx_v, rows0, rows1, rows2, outb_v, sem0, sem1, sem2
):
    wid = lax.axis_index("s") * NC + lax.axis_index("c")
    base = wid * B_PER_W

    pltpu.sync_copy(x_hbm.at[pl.ds(base * L, B_PER_W * L)], idx_v)

    def fire(rowbuf, sem, r):
        for off, n in CHUNKS:
            pltpu.async_copy(
                emb_hbm.at[idx_v.at[pl.ds(r * L + off, n)]],
                rowbuf.at[pl.ds(off, n)],
                sem,
            )

    def drain(rowbuf, sem, r):
        for off, n in CHUNKS:
            pltpu.make_async_copy(
                emb_hbm.at[idx_v.at[pl.ds(r * L + off, n)]],
                rowbuf.at[pl.ds(off, n)],
                sem,
            ).wait()

    def reduce(rowbuf, r):
        def red(k, accs):
            new = accs
            for u in range(RED_UNROLL):
                row = k * RED_UNROLL + u
                new = tuple(
                    new[c] + rowbuf[row, pl.ds(c * LANES, LANES)]
                    for c in range(VPR)
                )
            return new

        accs = lax.fori_loop(
            0,
            L // RED_UNROLL,
            red,
            tuple(jnp.zeros((LANES,), jnp.float32) for _ in range(VPR)),
        )
        for c in range(VPR):
            outb_v[r, pl.ds(c * LANES, LANES)] = accs[c] * (1.0 / L)

    bufs = (rows0, rows1, rows2)
    sems = (sem0, sem1, sem2)
    NBUF = 3
    fire(rows0, sem0, 0)
    fire(rows1, sem1, 1)
    fire(rows2, sem2, 2)

    NFULL = B_PER_W // NBUF

    def body(i, _):
        r0 = NBUF * i
        for s in range(NBUF):
            drain(bufs[s], sems[s], r0 + s)
            reduce(bufs[s], r0 + s)

            @pl.when(r0 + s + NBUF < B_PER_W)
            def _():
                fire(bufs[s], sems[s], r0 + s + NBUF)

        return 0

    lax.fori_loop(0, NFULL, body, 0)
    for r in range(NFULL * NBUF, B_PER_W):
        drain(bufs[r % NBUF], sems[r % NBUF], r)
        reduce(bufs[r % NBUF], r)
    pltpu.sync_copy(outb_v, out_hbm.at[pl.ds(base, B_PER_W)])


def _sc_pool(x_flat, emb):
    mesh = plsc.VectorSubcoreMesh(
        core_axis_name="c", subcore_axis_name="s", num_cores=NC, num_subcores=NS
    )
    return pl.kernel(
        _pool_body,
        out_type=jax.ShapeDtypeStruct((B, EMBED), jnp.float32),
        mesh=mesh,
        scratch_types=[
            pltpu.VMEM((B_PER_W * L,), jnp.int32),
            pltpu.VMEM((L, EMBED), jnp.float32),
            pltpu.VMEM((L, EMBED), jnp.float32),
            pltpu.VMEM((L, EMBED), jnp.float32),
            pltpu.VMEM((B_PER_W, EMBED), jnp.float32),
            pltpu.SemaphoreType.DMA,
            pltpu.SemaphoreType.DMA,
            pltpu.SemaphoreType.DMA,
        ],
    )(x_flat, emb)


def _mlp_body(p_ref, w1_ref, b1_ref, w2_ref, b2_ref, o_ref):
    h = jnp.dot(p_ref[...], w1_ref[...], preferred_element_type=jnp.float32)
    h = jnp.maximum(h + b1_ref[...], 0.0)
    o_ref[...] = (
        jnp.dot(h, w2_ref[...], preferred_element_type=jnp.float32) + b2_ref[...]
    )


def _tc_mlp(pooled, W1, b1, W2, b2):
    blk = 512
    grid = B // blk
    return pl.pallas_call(
        _mlp_body,
        grid=(grid,),
        in_specs=[
            pl.BlockSpec((blk, EMBED), lambda i: (i, 0)),
            pl.BlockSpec((EMBED, HIDDEN), lambda i: (0, 0)),
            pl.BlockSpec((1, HIDDEN), lambda i: (0, 0)),
            pl.BlockSpec((HIDDEN, NUM_CLASSES), lambda i: (0, 0)),
            pl.BlockSpec((1, NUM_CLASSES), lambda i: (0, 0)),
        ],
        out_specs=pl.BlockSpec((blk, NUM_CLASSES), lambda i: (i, 0)),
        out_shape=jax.ShapeDtypeStruct((B, NUM_CLASSES), jnp.float32),
    )(pooled, W1, b1.reshape(1, HIDDEN), W2, b2.reshape(1, NUM_CLASSES))


def kernel(x, emb, W1, b1, W2, b2):
    x_flat = x.reshape(-1).astype(jnp.int32)
    pooled = _sc_pool(x_flat, emb)
    return _tc_mlp(pooled, W1, b1, W2, b2)
